# baseline (speedup 1.0000x reference)
import functools

import jax
import jax.numpy as jnp
from jax import lax
from jax.experimental import pallas as pl
from jax.experimental.pallas import tpu as pltpu
from jax.experimental.pallas import tpu_sc as plsc

N = 10000
NP = 10240
E = 320000
EP = 327680
IN = 128
OUT = 64
H = 8
EH = 16
HID = 128
NG = 4
CH = 128
SUB = 8
BIG = SUB * CH
NB = EP // BIG
BPW = NB // 32
SLAB = NP // 16
BN = 400
GRID = N // BN
BNP = 512
GRIDP = NP // BNP
RPW = NP // 32
LMAX = 11264
LC = LMAX // CH

_mesh = plsc.VectorSubcoreMesh(core_axis_name="c", subcore_axis_name="s")


def _worker_id():
    return lax.axis_index("s") * 2 + lax.axis_index("c")


@functools.partial(
    pl.kernel,
    out_type=(
        jax.ShapeDtypeStruct((EP, EH), jnp.float32),
        jax.ShapeDtypeStruct((EP, EH), jnp.float32),
        jax.ShapeDtypeStruct((2, NP, EH), jnp.float32),
        jax.ShapeDtypeStruct((2, NP, EH), jnp.float32),
    ),
    mesh=_mesh,
    compiler_params=pltpu.CompilerParams(use_tc_tiling_on_sc=False),
    scratch_types=[
        pltpu.VMEM((SUB, CH), jnp.int32),
        pltpu.VMEM((SUB, CH), jnp.int32),
        pltpu.VMEM((BIG, EH), jnp.float32),
        pltpu.VMEM((BIG, EH), jnp.float32),
        pltpu.VMEM((BIG, EH), jnp.float32),
        pltpu.VMEM_SHARED((NP, EH), jnp.float32),
        pltpu.SemaphoreType.DMA,
        pltpu.SemaphoreType.DMA,
    ],
)
def _sc_phase1(src_0, dst_0, el_0, er_0, src_1, dst_1, el_1, er_1, zeros16,
               ex_0, ex_1, part_0, part_1,
               srcv, dstv, elb, erb, exb, acc, sem_a, sem_b):
    scid = lax.axis_index("c")
    sid = lax.axis_index("s")
    w = _worker_id()
    rows = pl.ds(sid * SLAB, SLAB)

    for src_h, dst_h, el_h, er_h, ex_h, part_h in (
        (src_0, dst_0, el_0, er_0, ex_0, part_0),
        (src_1, dst_1, el_1, er_1, ex_1, part_1),
    ):
        pltpu.sync_copy(zeros16.at[rows], acc.at[rows])
        plsc.subcore_barrier()

        def chunk_body(i, carry, src_h=src_h, dst_h=dst_h, el_h=el_h,
                       er_h=er_h, ex_h=ex_h):
            b = BPW * w + i
            pltpu.sync_copy(src_h.at[b], srcv)
            pltpu.sync_copy(dst_h.at[b], dstv)
            cps = []
            for k in range(SUB):
                sl = pl.ds(k * CH, CH)
                cps.append(
                    pltpu.async_copy(el_h.at[srcv.at[k]], elb.at[sl], sem_a))
                cps.append(
                    pltpu.async_copy(er_h.at[dstv.at[k]], erb.at[sl], sem_b))
            for c in cps:
                c.wait()

            def edge_body(e, c2):
                v = elb[e, :] + erb[e, :]
                v = jnp.where(v >= 0.0, v, 0.2 * v)
                exb[e, :] = jnp.exp(v)
                return c2

            lax.fori_loop(0, BIG, edge_body, 0, unroll=4)
            pltpu.sync_copy(exb, ex_h.at[pl.ds(b * BIG, BIG)])
            cps = []
            for k in range(SUB):
                sl = pl.ds(k * CH, CH)
                cps.append(pltpu.async_copy(
                    exb.at[sl], acc.at[dstv.at[k]], sem_b, add=True))
            for c in cps:
                c.wait()
            return carry

        lax.fori_loop(0, BPW, chunk_body, 0)
        plsc.subcore_barrier()
        pltpu.sync_copy(acc.at[rows], part_h.at[scid].at[rows])
        plsc.subcore_barrier()


@functools.partial(
    pl.kernel,
    out_type=(
        jax.ShapeDtypeStruct((EP, EH), jnp.float32),
        jax.ShapeDtypeStruct((EP, EH), jnp.float32),
    ),
    mesh=_mesh,
    compiler_params=pltpu.CompilerParams(use_tc_tiling_on_sc=False),
    scratch_types=[
        pltpu.VMEM((SUB, CH), jnp.int32),
        pltpu.VMEM((BIG, EH), jnp.float32),
        pltpu.VMEM((BIG, EH), jnp.float32),
        pltpu.SemaphoreType.DMA,
    ],
)
def _sc_phase2(dst_0, ex_0, rd_0, dst_1, ex_1, rd_1,
               att_0, att_1, dstv, exb, rdb, sem):
    w = _worker_id()

    for dst_h, ex_h, rd_h, att_h in (
        (dst_0, ex_0, rd_0, att_0),
        (dst_1, ex_1, rd_1, att_1),
    ):
        def chunk_body(i, carry, dst_h=dst_h, ex_h=ex_h, rd_h=rd_h,
                       att_h=att_h):
            b = BPW * w + i
            pltpu.sync_copy(dst_h.at[b], dstv)
            pltpu.sync_copy(ex_h.at[pl.ds(b * BIG, BIG)], exb)
            cps = []
            for k in range(SUB):
                sl = pl.ds(k * CH, CH)
                cps.append(
                    pltpu.async_copy(rd_h.at[dstv.at[k]], rdb.at[sl], sem))
            for c in cps:
                c.wait()

            def edge_body(e, c2):
                exb[e, :] = exb[e, :] * rdb[e, :]
                return c2

            lax.fori_loop(0, BIG, edge_body, 0, unroll=4)
            pltpu.sync_copy(exb, att_h.at[pl.ds(b * BIG, BIG)])
            return carry

        lax.fori_loop(0, BPW, chunk_body, 0)


@functools.partial(
    pl.kernel,
    out_type=tuple(
        jax.ShapeDtypeStruct((32, LMAX), jnp.int32) for _ in range(6)
    ),
    mesh=_mesh,
    compiler_params=pltpu.CompilerParams(
        use_tc_tiling_on_sc=False, needs_layout_passes=False),
    scratch_types=[
        pltpu.VMEM((SUB, CH), jnp.int32),
        pltpu.VMEM((SUB, CH), jnp.int32),
        pltpu.VMEM((LMAX,), jnp.int32),
        pltpu.VMEM((LMAX,), jnp.int32),
        pltpu.VMEM((LMAX,), jnp.int32),
    ],
)
def _sc_bin(src_0, dst_0, src_1, dst_1,
            sl_0, il_0, dl_0, sl_1, il_1, dl_1,
            srcb, dstb, sl, il, dl):
    w = _worker_id()
    lo = w * RPW
    lanes = jnp.arange(16, dtype=jnp.int32)

    for src_h, dst_h, sl_h, il_h, dl_h in (
        (src_0, dst_0, sl_0, il_0, dl_0),
        (src_1, dst_1, sl_1, il_1, dl_1),
    ):
        def fill_body(q, c2):
            s = pl.ds(q * 16, 16)
            sl[s] = jnp.full((16,), N, jnp.int32)
            il[s] = jnp.zeros((16,), jnp.int32)
            dl[s] = jnp.zeros((16,), jnp.int32)
            return c2

        lax.fori_loop(0, LMAX // 16, fill_body, 0)

        def big_body(b, cnt, src_h=src_h, dst_h=dst_h):
            pltpu.sync_copy(src_h.at[b], srcb)
            pltpu.sync_copy(dst_h.at[b], dstb)

            def vreg_body(q, cnt2):
                r = q >> 3
                m = q & 7
                s = pl.ds(m * 16, 16)
                dv = dstb[r, s]
                sv = srcb[r, s]
                iv = b * BIG + q * 16 + lanes
                mask = (dv >= lo) & (dv < lo + RPW)
                r = jnp.where(mask, 1, 0).astype(jnp.int32)
                for sh in (1, 2, 4, 8):
                    idx = jnp.maximum(lanes - sh, 0)
                    r = r + jnp.where(lanes >= sh, jnp.take(r, idx), 0)
                pos = cnt2 + r - 1
                plsc.store_scatter(sl, [pos], sv, mask=mask)
                plsc.store_scatter(il, [pos], iv, mask=mask)
                plsc.store_scatter(dl, [pos], dv - lo, mask=mask)
                npop = plsc.all_reduce_population_count(mask)
                return cnt2 + npop[0]

            return lax.fori_loop(0, SUB * CH // 16, vreg_body, cnt)

        lax.fori_loop(0, NB, big_body, 0)
        pltpu.sync_copy(sl, sl_h.at[w])
        pltpu.sync_copy(il, il_h.at[w])
        pltpu.sync_copy(dl, dl_h.at[w])


@functools.partial(
    pl.kernel,
    out_type=(
        jax.ShapeDtypeStruct((NG, NP, IN), jnp.float32),
        jax.ShapeDtypeStruct((NG, NP, IN), jnp.float32),
    ),
    mesh=_mesh,
    compiler_params=pltpu.CompilerParams(use_tc_tiling_on_sc=False),
    scratch_types=[
        pltpu.VMEM((2, CH), jnp.int32),
        pltpu.VMEM((2, CH), jnp.int32),
        pltpu.VMEM((2, CH), jnp.int32),
        pltpu.VMEM((2, CH, EH), jnp.float32),
        pltpu.VMEM((CH, IN), jnp.float32),
        pltpu.VMEM((CH, IN), jnp.float32),
        pltpu.VMEM((RPW, IN), jnp.float32),
        pltpu.SemaphoreType.DMA,
        pltpu.SemaphoreType.DMA,
    ],
)
def _sc_phase3(sl_0, il_0, dl_0, att_0, feat_0, sl_1, il_1, dl_1, att_1,
               feat_1, zeros128, out_0, out_1,
               srcv, idv, dlv, alb, fb0, fb1, acc, sem_g, sem_a):
    w = _worker_id()
    myrows = pl.ds(w * RPW, RPW)

    for sl_h, il_h, dl_h, att_h, feat_h, out_h in (
        (sl_0, il_0, dl_0, att_0, feat_0, out_0),
        (sl_1, il_1, dl_1, att_1, feat_1, out_1),
    ):
        for hg in range(NG):
            pltpu.sync_copy(zeros128.at[pl.ds(0, RPW)], acc)

            fbs = (fb0, fb1)

            def pair_body(i, carry, sl_h=sl_h, il_h=il_h, dl_h=dl_h,
                          att_h=att_h, feat_h=feat_h, hg=hg):
                gf = [None, None]
                ga = [None, None]
                for u in range(2):
                    ce = pl.ds((2 * i + u) * CH, CH)
                    pltpu.sync_copy(sl_h.at[w, ce], srcv.at[u])
                    pltpu.sync_copy(il_h.at[w, ce], idv.at[u])
                    pltpu.sync_copy(dl_h.at[w, ce], dlv.at[u])
                    ga[u] = pltpu.async_copy(
                        att_h.at[idv.at[u]], alb.at[u], sem_a)
                    gf[u] = pltpu.async_copy(
                        feat_h.at[hg].at[srcv.at[u]], fbs[u], sem_g)

                for u in range(2):
                    ga[u].wait()
                    gf[u].wait()
                    fb = fbs[u]

                    def vreg_body(q, c2, hg=hg, u=u, fb=fb):
                        dv = dlv[u, pl.ds(q * 16, 16)]
                        for lane in range(16):
                            e = q * 16 + lane
                            dstl = dv[lane]
                            av = alb[u, e, :]
                            a0 = jnp.broadcast_to(av[2 * hg], (16,))
                            a1 = jnp.broadcast_to(av[2 * hg + 1], (16,))
                            for j in range(8):
                                va = a0 if j < 4 else a1
                                sj = pl.ds(j * 16, 16)
                                plsc.addupdate(
                                    acc.at[dstl, sj], fb[e, sj] * va)
                        return c2

                    lax.fori_loop(0, CH // 16, vreg_body, 0)
                return carry

            lax.fori_loop(0, LC // 2, pair_body, 0)
            pltpu.sync_copy(acc, out_h.at[hg, myrows])


def _dot(a, b):
    return jax.lax.dot_general(a, b, (((1,), (0,)), ((), ())),
                               precision=jax.lax.Precision.HIGHEST,
                               preferred_element_type=jnp.float32)


def _tc_pre_body(h_ref, w0_ref, al0_ref, ar0_ref, w1_ref, al1_ref, ar1_ref,
                 f0_ref, el0_ref, er0_ref, f1_ref, el1_ref, er1_ref):
    hb = h_ref[...]
    for w_ref, al_ref, ar_ref, f_ref, el_ref, er_ref in (
        (w0_ref, al0_ref, ar0_ref, f0_ref, el0_ref, er0_ref),
        (w1_ref, al1_ref, ar1_ref, f1_ref, el1_ref, er1_ref),
    ):
        parts = []
        el = jnp.zeros((BNP, EH), jnp.float32)
        er = jnp.zeros((BNP, EH), jnp.float32)
        for g in range(NG):
            fg = _dot(hb, w_ref[:, g * IN:(g + 1) * IN])
            parts.append(fg)
            el = el + _dot(fg, al_ref[g * IN:(g + 1) * IN, :])
            er = er + _dot(fg, ar_ref[g * IN:(g + 1) * IN, :])
        f_ref[...] = jnp.stack(parts, axis=0)
        el_ref[...] = el
        er_ref[...] = er


def _tc_pre(hp, W_0, Al0, Ar0, W_1, Al1, Ar1):
    full = lambda s: pl.BlockSpec(s, lambda i: (0,) * len(s))
    row = lambda s: pl.BlockSpec(s, lambda i: (i,) + (0,) * (len(s) - 1))
    return pl.pallas_call(
        _tc_pre_body,
        grid=(GRIDP,),
        in_specs=[
            row((BNP, IN)),
            full((IN, H * OUT)), full((H * OUT, EH)), full((H * OUT, EH)),
            full((IN, H * OUT)), full((H * OUT, EH)), full((H * OUT, EH)),
        ],
        out_specs=[
            pl.BlockSpec((NG, BNP, IN), lambda i: (0, i, 0)),
            row((BNP, EH)), row((BNP, EH)),
            pl.BlockSpec((NG, BNP, IN), lambda i: (0, i, 0)),
            row((BNP, EH)), row((BNP, EH)),
        ],
        out_shape=[
            jax.ShapeDtypeStruct((NG, NP, IN), jnp.float32),
            jax.ShapeDtypeStruct((NP, EH), jnp.float32),
            jax.ShapeDtypeStruct((NP, EH), jnp.float32),
            jax.ShapeDtypeStruct((NG, NP, IN), jnp.float32),
            jax.ShapeDtypeStruct((NP, EH), jnp.float32),
            jax.ShapeDtypeStruct((NP, EH), jnp.float32),
        ],
    )(hp, W_0, Al0, Ar0, W_1, Al1, Ar1)


def _tc_combine_body(p0_ref, p1_ref, r0_ref, r1_ref):
    r0_ref[...] = 1.0 / (p0_ref[0] + p0_ref[1] + 1e-9)
    r1_ref[...] = 1.0 / (p1_ref[0] + p1_ref[1] + 1e-9)


def _tc_combine(part0, part1):
    spec = pl.BlockSpec((2, BNP, EH), lambda i: (0, i, 0))
    rspec = pl.BlockSpec((BNP, EH), lambda i: (i, 0))
    return pl.pallas_call(
        _tc_combine_body,
        grid=(GRIDP,),
        in_specs=[spec, spec],
        out_specs=[rspec, rspec],
        out_shape=[jax.ShapeDtypeStruct((NP, EH), jnp.float32)] * 2,
    )(part0, part1)


def _tc_post1_body(p0_ref, p1_ref, b0_ref, b1_ref, w1_ref, sb1_ref, w2_ref,
                   z0_ref, z1_ref, ws_ref):
    i = pl.program_id(0)

    @pl.when(i == 0)
    def _():
        ws_ref[...] = jnp.zeros((8, 128), jnp.float32)

    ri = lax.broadcasted_iota(jnp.int32, (8, 128), 0)
    ci = lax.broadcasted_iota(jnp.int32, (8, 128), 1)
    acc = ws_ref[...]
    for p, (p_ref, b_ref, z_ref) in enumerate(
            ((p0_ref, b0_ref, z0_ref), (p1_ref, b1_ref, z1_ref))):
        zb = jnp.concatenate([p_ref[g] for g in range(NG)], axis=-1)
        zb = zb + b_ref[0][None, :]
        zb = jnp.where(zb > 0.0, zb, jnp.exp(jnp.minimum(zb, 0.0)) - 1.0)
        z_ref[...] = zb
        t = jnp.tanh(_dot(zb, w1_ref[...]) + sb1_ref[0][None, :])
        s = jnp.sum(t * w2_ref[0][None, :])
        acc = acc + jnp.where((ri == p) & (ci == 0), s, 0.0)
    ws_ref[...] = acc


def _tc_post1(part3_0, part3_1, b0, b1, semW1, semb1, w2row):
    pspec = pl.BlockSpec((NG, BN, IN), lambda i: (0, i, 0))
    full = lambda s: pl.BlockSpec(s, lambda i: (0,) * len(s))
    zspec = pl.BlockSpec((BN, H * OUT), lambda i: (i, 0))
    return pl.pallas_call(
        _tc_post1_body,
        grid=(GRID,),
        in_specs=[pspec, pspec, full((8, H * OUT)), full((8, H * OUT)),
                  full((H * OUT, HID)), full((8, HID)), full((8, HID))],
        out_specs=[zspec, zspec, full((8, 128))],
        out_shape=[
            jax.ShapeDtypeStruct((N, H * OUT), jnp.float32),
            jax.ShapeDtypeStruct((N, H * OUT), jnp.float32),
            jax.ShapeDtypeStruct((8, 128), jnp.float32),
        ],
    )(part3_0, part3_1, b0, b1, semW1, semb1, w2row)


def _tc_post2_body(ws_ref, z0_ref, z1_ref, gat_ref, beta_ref):
    i = pl.program_id(0)
    w0 = ws_ref[0, 0] / N
    w1 = ws_ref[1, 0] / N
    m = jnp.maximum(w0, w1)
    e0 = jnp.exp(w0 - m)
    e1 = jnp.exp(w1 - m)
    b0 = e0 / (e0 + e1)
    b1 = e1 / (e0 + e1)
    gat_ref[...] = b0 * z0_ref[...] + b1 * z1_ref[...]

    @pl.when(i == 0)
    def _():
        ri = lax.broadcasted_iota(jnp.int32, (8, 128), 0)
        ci = lax.broadcasted_iota(jnp.int32, (8, 128), 1)
        beta_ref[...] = (jnp.where((ri == 0) & (ci == 0), b0, 0.0)
                         + jnp.where((ri == 1) & (ci == 0), b1, 0.0))


def _tc_post2(wsum, z0, z1):
    full = lambda s: pl.BlockSpec(s, lambda i: (0,) * len(s))
    zspec = pl.BlockSpec((BN, H * OUT), lambda i: (i, 0))
    return pl.pallas_call(
        _tc_post2_body,
        grid=(GRID,),
        in_specs=[full((8, 128)), zspec, zspec],
        out_specs=[zspec, full((8, 128))],
        out_shape=[
            jax.ShapeDtypeStruct((N, H * OUT), jnp.float32),
            jax.ShapeDtypeStruct((8, 128), jnp.float32),
        ],
    )(wsum, z0, z1)


def _fold_attn(a):
    eye = jnp.eye(H, EH, dtype=jnp.float32)
    return (a[:, :, None] * eye[:, None, :]).reshape(H * OUT, EH)


def _pad_edges(ei):
    spad = jnp.full((EP - E,), N, dtype=ei.dtype)
    dpad = N + (jnp.arange(EP - E, dtype=ei.dtype) % (NP - N))
    srcp = jnp.concatenate([ei[0], spad])
    dstp = jnp.concatenate([ei[1], dpad])
    return srcp.reshape(NB, SUB, CH), dstp.reshape(NB, SUB, CH)


def kernel(h, edge_index_0, edge_index_1, W_0, a_l_0, a_r_0, b_0,
           W_1, a_l_1, a_r_1, b_1, sem_W1, sem_b1, sem_W2):
    src0, dst0 = _pad_edges(edge_index_0)
    src1, dst1 = _pad_edges(edge_index_1)
    Al0, Ar0 = _fold_attn(a_l_0), _fold_attn(a_r_0)
    Al1, Ar1 = _fold_attn(a_l_1), _fold_attn(a_r_1)
    zeros16 = jnp.zeros((NP, EH), jnp.float32)
    zeros128 = jnp.zeros((NP, IN), jnp.float32)
    hp = jnp.concatenate(
        [h, jnp.zeros((NP - N, IN), jnp.float32)], axis=0)

    feat0, el0, er0, feat1, el1, er1 = _tc_pre(
        hp, W_0, Al0, Ar0, W_1, Al1, Ar1)

    ex0, ex1, part0, part1 = _sc_phase1(
        src0, dst0, el0, er0, src1, dst1, el1, er1, zeros16)
    rd0, rd1 = _tc_combine(part0, part1)
    att0, att1 = _sc_phase2(dst0, ex0, rd0, dst1, ex1, rd1)
    sl0, il0, dl0, sl1, il1, dl1 = _sc_bin(src0, dst0, src1, dst1)
    p3_0, p3_1 = _sc_phase3(
        sl0, il0, dl0, att0, feat0, sl1, il1, dl1, att1, feat1, zeros128)

    b0b = jnp.broadcast_to(b_0[None, :], (8, H * OUT))
    b1b = jnp.broadcast_to(b_1[None, :], (8, H * OUT))
    sb1 = jnp.broadcast_to(sem_b1[None, :], (8, HID))
    w2r = jnp.broadcast_to(sem_W2.T, (8, HID))
    z0, z1, wsum = _tc_post1(p3_0, p3_1, b0b, b1b, sem_W1, sb1, w2r)
    gat_out, beta_pad = _tc_post2(wsum, z0, z1)

    return (gat_out, att0[:E, :H, None], att1[:E, :H, None],
            beta_pad[:2, :1])


# speedup vs baseline: 2.4254x; 2.4254x over previous
import functools

import jax
import jax.numpy as jnp
from jax import lax
from jax.experimental import pallas as pl
from jax.experimental.pallas import tpu as pltpu
from jax.experimental.pallas import tpu_sc as plsc

N = 10000
NP = 10240
E = 320000
EP = 327680
IN = 128
OUT = 64
H = 8
EH = 16
HID = 128
NG = 4
CH = 128
SUB = 8
BIG = SUB * CH
NB = EP // BIG
BPW = NB // 32
SLAB = NP // 16
BN = 400
GRID = N // BN
BNP = 512
GRIDP = NP // BNP

_mesh = plsc.VectorSubcoreMesh(core_axis_name="c", subcore_axis_name="s")


def _worker_id():
    return lax.axis_index("s") * 2 + lax.axis_index("c")


@functools.partial(
    pl.kernel,
    out_type=(
        jax.ShapeDtypeStruct((EP, EH), jnp.float32),
        jax.ShapeDtypeStruct((EP, EH), jnp.float32),
        jax.ShapeDtypeStruct((2, NP, EH), jnp.float32),
        jax.ShapeDtypeStruct((2, NP, EH), jnp.float32),
    ),
    mesh=_mesh,
    compiler_params=pltpu.CompilerParams(use_tc_tiling_on_sc=False),
    scratch_types=[
        pltpu.VMEM((SUB, CH), jnp.int32),
        pltpu.VMEM((SUB, CH), jnp.int32),
        pltpu.VMEM((BIG, EH), jnp.float32),
        pltpu.VMEM((BIG, EH), jnp.float32),
        pltpu.VMEM((BIG, EH), jnp.float32),
        pltpu.VMEM_SHARED((NP, EH), jnp.float32),
        pltpu.SemaphoreType.DMA,
        pltpu.SemaphoreType.DMA,
    ],
)
def _sc_phase1(src_0, dst_0, el_0, er_0, src_1, dst_1, el_1, er_1, zeros16,
               ex_0, ex_1, part_0, part_1,
               srcv, dstv, elb, erb, exb, acc, sem_a, sem_b):
    scid = lax.axis_index("c")
    sid = lax.axis_index("s")
    w = _worker_id()
    rows = pl.ds(sid * SLAB, SLAB)

    for src_h, dst_h, el_h, er_h, ex_h, part_h in (
        (src_0, dst_0, el_0, er_0, ex_0, part_0),
        (src_1, dst_1, el_1, er_1, ex_1, part_1),
    ):
        pltpu.sync_copy(zeros16.at[rows], acc.at[rows])
        plsc.subcore_barrier()

        def chunk_body(i, carry, src_h=src_h, dst_h=dst_h, el_h=el_h,
                       er_h=er_h, ex_h=ex_h):
            b = BPW * w + i
            pltpu.sync_copy(src_h.at[b], srcv)
            pltpu.sync_copy(dst_h.at[b], dstv)
            cps = []
            for k in range(SUB):
                sl = pl.ds(k * CH, CH)
                cps.append(
                    pltpu.async_copy(el_h.at[srcv.at[k]], elb.at[sl], sem_a))
                cps.append(
                    pltpu.async_copy(er_h.at[dstv.at[k]], erb.at[sl], sem_b))
            for c in cps:
                c.wait()

            def edge_body(e, c2):
                v = elb[e, :] + erb[e, :]
                v = jnp.where(v >= 0.0, v, 0.2 * v)
                exb[e, :] = jnp.exp(v)
                return c2

            lax.fori_loop(0, BIG, edge_body, 0, unroll=4)
            pltpu.sync_copy(exb, ex_h.at[pl.ds(b * BIG, BIG)])
            cps = []
            for k in range(SUB):
                sl = pl.ds(k * CH, CH)
                cps.append(pltpu.async_copy(
                    exb.at[sl], acc.at[dstv.at[k]], sem_b, add=True))
            for c in cps:
                c.wait()
            return carry

        lax.fori_loop(0, BPW, chunk_body, 0)
        plsc.subcore_barrier()
        pltpu.sync_copy(acc.at[rows], part_h.at[scid].at[rows])
        plsc.subcore_barrier()


@functools.partial(
    pl.kernel,
    out_type=(
        jax.ShapeDtypeStruct((EP, EH), jnp.float32),
        jax.ShapeDtypeStruct((EP, EH), jnp.float32),
    ),
    mesh=_mesh,
    compiler_params=pltpu.CompilerParams(use_tc_tiling_on_sc=False),
    scratch_types=[
        pltpu.VMEM((SUB, CH), jnp.int32),
        pltpu.VMEM((BIG, EH), jnp.float32),
        pltpu.VMEM((BIG, EH), jnp.float32),
        pltpu.SemaphoreType.DMA,
    ],
)
def _sc_phase2(dst_0, ex_0, rd_0, dst_1, ex_1, rd_1,
               att_0, att_1, dstv, exb, rdb, sem):
    w = _worker_id()

    for dst_h, ex_h, rd_h, att_h in (
        (dst_0, ex_0, rd_0, att_0),
        (dst_1, ex_1, rd_1, att_1),
    ):
        def chunk_body(i, carry, dst_h=dst_h, ex_h=ex_h, rd_h=rd_h,
                       att_h=att_h):
            b = BPW * w + i
            pltpu.sync_copy(dst_h.at[b], dstv)
            pltpu.sync_copy(ex_h.at[pl.ds(b * BIG, BIG)], exb)
            cps = []
            for k in range(SUB):
                sl = pl.ds(k * CH, CH)
                cps.append(
                    pltpu.async_copy(rd_h.at[dstv.at[k]], rdb.at[sl], sem))
            for c in cps:
                c.wait()

            def edge_body(e, c2):
                exb[e, :] = exb[e, :] * rdb[e, :]
                return c2

            lax.fori_loop(0, BIG, edge_body, 0, unroll=4)
            pltpu.sync_copy(exb, att_h.at[pl.ds(b * BIG, BIG)])
            return carry

        lax.fori_loop(0, BPW, chunk_body, 0)


@functools.partial(
    pl.kernel,
    out_type=(
        jax.ShapeDtypeStruct((NG, 2, NP, IN), jnp.float32),
        jax.ShapeDtypeStruct((NG, 2, NP, IN), jnp.float32),
    ),
    mesh=_mesh,
    compiler_params=pltpu.CompilerParams(use_tc_tiling_on_sc=False),
    scratch_types=[
        pltpu.VMEM((SUB, CH), jnp.int32),
        pltpu.VMEM((SUB, CH), jnp.int32),
        pltpu.VMEM((CH, EH), jnp.float32),
        pltpu.VMEM((CH, IN), jnp.float32),
        pltpu.VMEM((CH, IN), jnp.float32),
        pltpu.VMEM_SHARED((NP, IN), jnp.float32),
        pltpu.SemaphoreType.DMA,
        pltpu.SemaphoreType.DMA,
    ],
)
def _sc_phase3(src_0, dst_0, att_0, feat_0, src_1, dst_1, att_1, feat_1,
               zeros128, out_0, out_1, srcv, dstv, alb, fb0, fb1, acc,
               sem_g, sem_s):
    scid = lax.axis_index("c")
    sid = lax.axis_index("s")
    w = _worker_id()
    rows = pl.ds(sid * SLAB, SLAB)

    for src_h, dst_h, att_h, feat_h, out_h in (
        (src_0, dst_0, att_0, feat_0, out_0),
        (src_1, dst_1, att_1, feat_1, out_1),
    ):
        for hg in range(NG):
            pltpu.sync_copy(zeros128.at[rows], acc.at[rows])
            plsc.subcore_barrier()

            def chunk_body(i, carry, src_h=src_h, dst_h=dst_h, att_h=att_h,
                           feat_h=feat_h, hg=hg):
                b = BPW * w + i
                pltpu.sync_copy(src_h.at[b], srcv)
                pltpu.sync_copy(dst_h.at[b], dstv)
                bufs = (fb0, fb1)
                gathers = [None] * SUB
                scatters = [None] * SUB
                gathers[0] = pltpu.async_copy(
                    feat_h.at[hg].at[srcv.at[0]], bufs[0], sem_g)
                for k in range(SUB):
                    fb = bufs[k % 2]
                    pltpu.sync_copy(
                        att_h.at[pl.ds(b * BIG + k * CH, CH)], alb)
                    gathers[k].wait()
                    if k + 1 < SUB:
                        if k >= 1:
                            scatters[k - 1].wait()
                        gathers[k + 1] = pltpu.async_copy(
                            feat_h.at[hg].at[srcv.at[k + 1]],
                            bufs[(k + 1) % 2], sem_g)

                    def edge_body(e, c2, hg=hg, fb=fb):
                        av = alb[e, :]
                        a0 = jnp.broadcast_to(av[2 * hg], (16,))
                        a1 = jnp.broadcast_to(av[2 * hg + 1], (16,))
                        for j in range(8):
                            va = a0 if j < 4 else a1
                            sl = pl.ds(j * 16, 16)
                            fb[e, sl] = fb[e, sl] * va
                        return c2

                    lax.fori_loop(0, CH, edge_body, 0, unroll=2)
                    scatters[k] = pltpu.async_copy(
                        fb, acc.at[dstv.at[k]], sem_s, add=True)
                scatters[SUB - 2].wait()
                scatters[SUB - 1].wait()
                return carry

            lax.fori_loop(0, BPW, chunk_body, 0)
            plsc.subcore_barrier()
            pltpu.sync_copy(acc.at[rows], out_h.at[hg, scid].at[rows])
            plsc.subcore_barrier()


def _dot(a, b):
    return jax.lax.dot_general(a, b, (((1,), (0,)), ((), ())),
                               precision=jax.lax.Precision.HIGHEST,
                               preferred_element_type=jnp.float32)


def _tc_pre_body(h_ref, w0_ref, al0_ref, ar0_ref, w1_ref, al1_ref, ar1_ref,
                 f0_ref, el0_ref, er0_ref, f1_ref, el1_ref, er1_ref):
    hb = h_ref[...]
    for w_ref, al_ref, ar_ref, f_ref, el_ref, er_ref in (
        (w0_ref, al0_ref, ar0_ref, f0_ref, el0_ref, er0_ref),
        (w1_ref, al1_ref, ar1_ref, f1_ref, el1_ref, er1_ref),
    ):
        parts = []
        el = jnp.zeros((BNP, EH), jnp.float32)
        er = jnp.zeros((BNP, EH), jnp.float32)
        for g in range(NG):
            fg = _dot(hb, w_ref[:, g * IN:(g + 1) * IN])
            parts.append(fg)
            el = el + _dot(fg, al_ref[g * IN:(g + 1) * IN, :])
            er = er + _dot(fg, ar_ref[g * IN:(g + 1) * IN, :])
        f_ref[...] = jnp.stack(parts, axis=0)
        el_ref[...] = el
        er_ref[...] = er


def _tc_pre(hp, W_0, Al0, Ar0, W_1, Al1, Ar1):
    full = lambda s: pl.BlockSpec(s, lambda i: (0,) * len(s))
    row = lambda s: pl.BlockSpec(s, lambda i: (i,) + (0,) * (len(s) - 1))
    return pl.pallas_call(
        _tc_pre_body,
        grid=(GRIDP,),
        in_specs=[
            row((BNP, IN)),
            full((IN, H * OUT)), full((H * OUT, EH)), full((H * OUT, EH)),
            full((IN, H * OUT)), full((H * OUT, EH)), full((H * OUT, EH)),
        ],
        out_specs=[
            pl.BlockSpec((NG, BNP, IN), lambda i: (0, i, 0)),
            row((BNP, EH)), row((BNP, EH)),
            pl.BlockSpec((NG, BNP, IN), lambda i: (0, i, 0)),
            row((BNP, EH)), row((BNP, EH)),
        ],
        out_shape=[
            jax.ShapeDtypeStruct((NG, NP, IN), jnp.float32),
            jax.ShapeDtypeStruct((NP, EH), jnp.float32),
            jax.ShapeDtypeStruct((NP, EH), jnp.float32),
            jax.ShapeDtypeStruct((NG, NP, IN), jnp.float32),
            jax.ShapeDtypeStruct((NP, EH), jnp.float32),
            jax.ShapeDtypeStruct((NP, EH), jnp.float32),
        ],
    )(hp, W_0, Al0, Ar0, W_1, Al1, Ar1)


def _tc_combine_body(p0_ref, p1_ref, r0_ref, r1_ref):
    r0_ref[...] = 1.0 / (p0_ref[0] + p0_ref[1] + 1e-9)
    r1_ref[...] = 1.0 / (p1_ref[0] + p1_ref[1] + 1e-9)


def _tc_combine(part0, part1):
    spec = pl.BlockSpec((2, BNP, EH), lambda i: (0, i, 0))
    rspec = pl.BlockSpec((BNP, EH), lambda i: (i, 0))
    return pl.pallas_call(
        _tc_combine_body,
        grid=(GRIDP,),
        in_specs=[spec, spec],
        out_specs=[rspec, rspec],
        out_shape=[jax.ShapeDtypeStruct((NP, EH), jnp.float32)] * 2,
    )(part0, part1)


def _tc_post1_body(p0_ref, p1_ref, b0_ref, b1_ref, w1_ref, sb1_ref, w2_ref,
                   z0_ref, z1_ref, ws_ref):
    i = pl.program_id(0)

    @pl.when(i == 0)
    def _():
        ws_ref[...] = jnp.zeros((8, 128), jnp.float32)

    ri = lax.broadcasted_iota(jnp.int32, (8, 128), 0)
    ci = lax.broadcasted_iota(jnp.int32, (8, 128), 1)
    acc = ws_ref[...]
    for p, (p_ref, b_ref, z_ref) in enumerate(
            ((p0_ref, b0_ref, z0_ref), (p1_ref, b1_ref, z1_ref))):
        zb = jnp.concatenate(
            [p_ref[g, 0] + p_ref[g, 1] for g in range(NG)], axis=-1)
        zb = zb + b_ref[0][None, :]
        zb = jnp.where(zb > 0.0, zb, jnp.exp(jnp.minimum(zb, 0.0)) - 1.0)
        z_ref[...] = zb
        t = jnp.tanh(_dot(zb, w1_ref[...]) + sb1_ref[0][None, :])
        s = jnp.sum(t * w2_ref[0][None, :])
        acc = acc + jnp.where((ri == p) & (ci == 0), s, 0.0)
    ws_ref[...] = acc


def _tc_post1(part3_0, part3_1, b0, b1, semW1, semb1, w2row):
    pspec = pl.BlockSpec((NG, 2, BN, IN), lambda i: (0, 0, i, 0))
    full = lambda s: pl.BlockSpec(s, lambda i: (0,) * len(s))
    zspec = pl.BlockSpec((BN, H * OUT), lambda i: (i, 0))
    return pl.pallas_call(
        _tc_post1_body,
        grid=(GRID,),
        in_specs=[pspec, pspec, full((8, H * OUT)), full((8, H * OUT)),
                  full((H * OUT, HID)), full((8, HID)), full((8, HID))],
        out_specs=[zspec, zspec, full((8, 128))],
        out_shape=[
            jax.ShapeDtypeStruct((N, H * OUT), jnp.float32),
            jax.ShapeDtypeStruct((N, H * OUT), jnp.float32),
            jax.ShapeDtypeStruct((8, 128), jnp.float32),
        ],
    )(part3_0, part3_1, b0, b1, semW1, semb1, w2row)


def _tc_post2_body(ws_ref, z0_ref, z1_ref, gat_ref, beta_ref):
    i = pl.program_id(0)
    w0 = ws_ref[0, 0] / N
    w1 = ws_ref[1, 0] / N
    m = jnp.maximum(w0, w1)
    e0 = jnp.exp(w0 - m)
    e1 = jnp.exp(w1 - m)
    b0 = e0 / (e0 + e1)
    b1 = e1 / (e0 + e1)
    gat_ref[...] = b0 * z0_ref[...] + b1 * z1_ref[...]

    @pl.when(i == 0)
    def _():
        ri = lax.broadcasted_iota(jnp.int32, (8, 128), 0)
        ci = lax.broadcasted_iota(jnp.int32, (8, 128), 1)
        beta_ref[...] = (jnp.where((ri == 0) & (ci == 0), b0, 0.0)
                         + jnp.where((ri == 1) & (ci == 0), b1, 0.0))


def _tc_post2(wsum, z0, z1):
    full = lambda s: pl.BlockSpec(s, lambda i: (0,) * len(s))
    zspec = pl.BlockSpec((BN, H * OUT), lambda i: (i, 0))
    return pl.pallas_call(
        _tc_post2_body,
        grid=(GRID,),
        in_specs=[full((8, 128)), zspec, zspec],
        out_specs=[zspec, full((8, 128))],
        out_shape=[
            jax.ShapeDtypeStruct((N, H * OUT), jnp.float32),
            jax.ShapeDtypeStruct((8, 128), jnp.float32),
        ],
    )(wsum, z0, z1)


def _fold_attn(a):
    eye = jnp.eye(H, EH, dtype=jnp.float32)
    return (a[:, :, None] * eye[:, None, :]).reshape(H * OUT, EH)


def _pad_edges(ei):
    pad = jnp.full((2, EP - E), N, dtype=ei.dtype)
    eip = jnp.concatenate([ei, pad], axis=1)
    return eip[0].reshape(NB, SUB, CH), eip[1].reshape(NB, SUB, CH)


def kernel(h, edge_index_0, edge_index_1, W_0, a_l_0, a_r_0, b_0,
           W_1, a_l_1, a_r_1, b_1, sem_W1, sem_b1, sem_W2):
    src0, dst0 = _pad_edges(edge_index_0)
    src1, dst1 = _pad_edges(edge_index_1)
    Al0, Ar0 = _fold_attn(a_l_0), _fold_attn(a_r_0)
    Al1, Ar1 = _fold_attn(a_l_1), _fold_attn(a_r_1)
    zeros16 = jnp.zeros((NP, EH), jnp.float32)
    zeros128 = jnp.zeros((NP, IN), jnp.float32)
    hp = jnp.concatenate(
        [h, jnp.zeros((NP - N, IN), jnp.float32)], axis=0)

    feat0, el0, er0, feat1, el1, er1 = _tc_pre(
        hp, W_0, Al0, Ar0, W_1, Al1, Ar1)

    ex0, ex1, part0, part1 = _sc_phase1(
        src0, dst0, el0, er0, src1, dst1, el1, er1, zeros16)
    rd0, rd1 = _tc_combine(part0, part1)
    att0, att1 = _sc_phase2(dst0, ex0, rd0, dst1, ex1, rd1)
    p3_0, p3_1 = _sc_phase3(
        src0, dst0, att0, feat0, src1, dst1, att1, feat1, zeros128)

    b0b = jnp.broadcast_to(b_0[None, :], (8, H * OUT))
    b1b = jnp.broadcast_to(b_1[None, :], (8, H * OUT))
    sb1 = jnp.broadcast_to(sem_b1[None, :], (8, HID))
    w2r = jnp.broadcast_to(sem_W2.T, (8, HID))
    z0, z1, wsum = _tc_post1(p3_0, p3_1, b0b, b1b, sem_W1, sb1, w2r)
    gat_out, beta_pad = _tc_post2(wsum, z0, z1)

    return (gat_out, att0[:E, :H, None], att1[:E, :H, None],
            beta_pad[:2, :1])
